# baseline (speedup 1.0000x reference)
import jax
import jax.numpy as jnp
from jax.experimental import pallas as pl
from jax.experimental.pallas import tpu as pltpu


def _bcast_body(w_ref, o_ref):
    o_ref[...] = jnp.broadcast_to(w_ref[0:1, :], o_ref.shape)


def kernel(input, weight):
    B, L = input.shape
    D = weight.shape[1]
    rows = B * L
    blk = 32768
    grid = rows // blk
    out = pl.pallas_call(
        _bcast_body,
        grid=(grid,),
        in_specs=[pl.BlockSpec((1, D), lambda i: (0, 0))],
        out_specs=pl.BlockSpec((blk, D), lambda i: (i, 0)),
        out_shape=jax.ShapeDtypeStruct((rows, D), jnp.float32),
        compiler_params=pltpu.CompilerParams(
            dimension_semantics=("arbitrary",),
        ),
    )(weight)
    return out.reshape(B, L, D)


# speedup vs baseline: 1.0078x; 1.0078x over previous
import jax
import jax.numpy as jnp
from jax.experimental import pallas as pl
from jax.experimental.pallas import tpu as pltpu

_SREP = 32768


def _dma_body(w_ref, o_ref, scratch_ref, sem):
    scratch_ref[...] = jnp.broadcast_to(w_ref[0:1, :], scratch_ref.shape)
    ncopy = o_ref.shape[0] // _SREP

    def start(i, carry):
        pltpu.make_async_copy(
            scratch_ref, o_ref.at[pl.ds(i * _SREP, _SREP), :], sem
        ).start()
        return carry

    jax.lax.fori_loop(0, ncopy, start, 0)

    def wait(i, carry):
        pltpu.make_async_copy(
            scratch_ref, o_ref.at[pl.ds(0, _SREP), :], sem
        ).wait()
        return carry

    jax.lax.fori_loop(0, ncopy, wait, 0)


def kernel(input, weight):
    B, L = input.shape
    D = weight.shape[1]
    rows = B * L
    out = pl.pallas_call(
        _dma_body,
        in_specs=[pl.BlockSpec(memory_space=pltpu.VMEM)],
        out_specs=pl.BlockSpec(memory_space=pltpu.MemorySpace.HBM),
        out_shape=jax.ShapeDtypeStruct((rows, D), jnp.float32),
        scratch_shapes=[
            pltpu.VMEM((_SREP, D), jnp.float32),
            pltpu.SemaphoreType.DMA,
        ],
    )(weight)
    return out.reshape(B, L, D)
